# baseline (speedup 1.0000x reference)
import jax
import jax.numpy as jnp
from jax.experimental import pallas as pl
from jax.experimental.pallas import tpu as pltpu

_B = 128
_V = 100000
_ROWS_PER_BLOCK = 8


def _scatter_softmax_kernel(gold_ref, x_ref, o_ref):
    i = pl.program_id(0)
    x = x_ref[...]
    base = i * _ROWS_PER_BLOCK
    golds = jnp.stack(
        [gold_ref[base + r] for r in range(_ROWS_PER_BLOCK)]
    ).reshape(_ROWS_PER_BLOCK, 1)
    col = jax.lax.broadcasted_iota(jnp.int32, x.shape, 1)
    y = jnp.where(col == golds, jnp.float32(_V), x)
    m = jnp.max(y, axis=1, keepdims=True)
    e = jnp.exp(y - m)
    s = jnp.sum(e, axis=1, keepdims=True)
    o_ref[...] = e / s


def kernel(t, gold):
    grid_spec = pltpu.PrefetchScalarGridSpec(
        num_scalar_prefetch=1,
        grid=(_B // _ROWS_PER_BLOCK,),
        in_specs=[
            pl.BlockSpec((_ROWS_PER_BLOCK, _V), lambda i, g: (i, 0)),
        ],
        out_specs=pl.BlockSpec((_ROWS_PER_BLOCK, _V), lambda i, g: (i, 0)),
    )
    return pl.pallas_call(
        _scatter_softmax_kernel,
        grid_spec=grid_spec,
        out_shape=jax.ShapeDtypeStruct((_B, _V), jnp.float32),
        compiler_params=pltpu.CompilerParams(
            dimension_semantics=("arbitrary",),
        ),
    )(gold, t)
